# baseline (speedup 1.0000x reference)
import jax
import jax.numpy as jnp
from jax import lax
from jax.experimental import pallas as pl
from jax.experimental.pallas import tpu as pltpu

_ALPHA = 0.2
_BLK_R = 256


def _lrelu(v):
    return jnp.maximum(v, _ALPHA * v)


def _elu(v):
    return jnp.where(v > 0, v, jnp.exp(jnp.minimum(v, 0.0)) - 1.0)


def _stgat_kernel(nhead, nhid, npred,
                  x_ref, adj_ref, wall_ref, a1_ref, a2_ref, a2t_ref,
                  wout_ref, a1ot_ref, a2ot_ref, out_ref,
                  va_s, e1_s, e2_s, f1t_s, f2t_s, fb1_s, h_s,
                  vap_s, e1p_s, e2p_s, f1pt_s, f2pt_s, fbp_s):
    ph = pl.program_id(1)
    nb = pl.program_id(2)

    @pl.when(jnp.logical_and(ph == 0, nb == 0))
    def _proj1():
        wh = jnp.dot(x_ref[0], wall_ref[...],
                     preferred_element_type=jnp.float32)
        wh1 = jnp.dot(wh, a1_ref[...],
                      preferred_element_type=jnp.float32)
        wh2 = jnp.dot(wh, a2_ref[...],
                      preferred_element_type=jnp.float32)
        wh2t = lax.dot_general(
            a2t_ref[...], wh, (((1,), (1,)), ((), ())),
            preferred_element_type=jnp.float32)
        m2 = jnp.max(wh2, axis=0, keepdims=True)
        c = _lrelu(wh1 + m2)
        e1_s[...] = jnp.exp(wh1 - c).astype(jnp.bfloat16)
        e2_s[...] = jnp.exp(_ALPHA * wh1 - c).astype(jnp.bfloat16)
        f1t_s[...] = jnp.exp(wh2t).astype(jnp.bfloat16)
        f2t_s[...] = jnp.exp(_ALPHA * wh2t).astype(jnp.bfloat16)
        ones_col = jnp.ones((wh.shape[0], 1), jnp.bfloat16)
        for h in range(nhead):
            whh = wh[:, h * nhid:(h + 1) * nhid]
            va_s[h, :, :nhid] = whh.astype(jnp.bfloat16)
            va_s[h, :, nhid:nhid + 1] = ones_col
            fb1_s[h:h + 1, :] = jnp.mean(whh, axis=0, keepdims=True)

    @pl.when(jnp.logical_and(ph == 1, nb == 0))
    def _proj2():
        whp = jnp.dot(h_s[...], wout_ref[...],
                      preferred_element_type=jnp.float32)
        wh1p = lax.dot_general(
            whp, a1ot_ref[...], (((1,), (1,)), ((), ())),
            preferred_element_type=jnp.float32)
        wh2pt = lax.dot_general(
            a2ot_ref[...], whp, (((1,), (1,)), ((), ())),
            preferred_element_type=jnp.float32)
        cp = _lrelu(wh1p + jnp.max(wh2pt))
        e1p_s[...] = jnp.exp(wh1p - cp).astype(jnp.bfloat16)
        e2p_s[...] = jnp.exp(_ALPHA * wh1p - cp).astype(jnp.bfloat16)
        f1pt_s[...] = jnp.exp(wh2pt).astype(jnp.bfloat16)
        f2pt_s[...] = jnp.exp(_ALPHA * wh2pt).astype(jnp.bfloat16)
        vap_s[:, :npred] = whp.astype(jnp.bfloat16)
        vap_s[:, npred:npred + 1] = jnp.ones((whp.shape[0], 1), jnp.bfloat16)
        fbp_s[...] = jnp.mean(whp, axis=0, keepdims=True)

    rows = pl.ds(nb * _BLK_R, _BLK_R)
    adjb = adj_ref[0, rows, :]

    @pl.when(ph == 0)
    def _layer1():
        e1b = e1_s[rows, :]
        e2b = e2_s[rows, :]
        outs = []
        for h in range(nhead):
            p = jnp.maximum(e1b[:, h:h + 1] * f1t_s[h:h + 1, :],
                            e2b[:, h:h + 1] * f2t_s[h:h + 1, :])
            w = adjb * p
            na = jnp.dot(w, va_s[h], preferred_element_type=jnp.float32)
            num = na[:, :nhid]
            den = na[:, nhid:nhid + 1]
            hp = jnp.where(den > 0, num / den, fb1_s[h:h + 1, :])
            outs.append(_elu(hp))
        h_s[rows, :] = jnp.concatenate(outs, axis=1)

    @pl.when(ph == 1)
    def _layer2():
        e1b = e1p_s[rows, :]
        e2b = e2p_s[rows, :]
        p = jnp.maximum(e1b * f1pt_s[...], e2b * f2pt_s[...])
        w = adjb * p
        na = jnp.dot(w, vap_s[...], preferred_element_type=jnp.float32)
        num = na[:, :npred]
        den = na[:, npred:npred + 1]
        hp = jnp.where(den > 0, num / den, fbp_s[...])
        out_ref[0] = _elu(hp)


def kernel(x, adj, Ws, a_s, W_out, a_out):
    B, N, F = x.shape
    nhead, _, nhid = Ws.shape
    npred = W_out.shape[1]
    H = nhead * nhid

    wall = jnp.transpose(Ws, (1, 0, 2)).reshape(F, H)
    eye = jnp.eye(nhead, dtype=x.dtype)
    a1 = a_s[:, :nhid, 0]
    a2 = a_s[:, nhid:, 0]
    a1bd = (eye[:, :, None] * a1[None, :, :]).reshape(nhead, H).T
    a2bd = (eye[:, :, None] * a2[None, :, :]).reshape(nhead, H).T
    a2t = a2bd.T
    a1ot = a_out[:npred].T
    a2ot = a_out[npred:].T

    grid = (B, 2, N // _BLK_R)

    def _idx_const(b, ph, nb):
        return (0, 0)

    out = pl.pallas_call(
        lambda *refs: _stgat_kernel(nhead, nhid, npred, *refs),
        grid=grid,
        in_specs=[
            pl.BlockSpec((1, N, F), lambda b, ph, nb: (b, 0, 0)),
            pl.BlockSpec((1, N, N), lambda b, ph, nb: (b, 0, 0)),
            pl.BlockSpec((F, H), _idx_const),
            pl.BlockSpec((H, nhead), _idx_const),
            pl.BlockSpec((H, nhead), _idx_const),
            pl.BlockSpec((nhead, H), _idx_const),
            pl.BlockSpec((H, npred), _idx_const),
            pl.BlockSpec((1, npred), _idx_const),
            pl.BlockSpec((1, npred), _idx_const),
        ],
        out_specs=pl.BlockSpec((1, _BLK_R, npred), lambda b, ph, nb: (b, nb, 0)),
        out_shape=jax.ShapeDtypeStruct((B, N, npred), jnp.float32),
        scratch_shapes=[
            pltpu.VMEM((nhead, N, nhid + 1), jnp.bfloat16),
            pltpu.VMEM((N, nhead), jnp.bfloat16),
            pltpu.VMEM((N, nhead), jnp.bfloat16),
            pltpu.VMEM((nhead, N), jnp.bfloat16),
            pltpu.VMEM((nhead, N), jnp.bfloat16),
            pltpu.VMEM((nhead, nhid), jnp.float32),
            pltpu.VMEM((N, H), jnp.float32),
            pltpu.VMEM((N, npred + 1), jnp.bfloat16),
            pltpu.VMEM((N, 1), jnp.bfloat16),
            pltpu.VMEM((N, 1), jnp.bfloat16),
            pltpu.VMEM((1, N), jnp.bfloat16),
            pltpu.VMEM((1, N), jnp.bfloat16),
            pltpu.VMEM((1, npred), jnp.float32),
        ],
    )(x, adj.astype(jnp.bfloat16), wall, a1bd, a2bd, a2t, W_out, a1ot, a2ot)
    return out.reshape(B, N * npred)


# speedup vs baseline: 1.0777x; 1.0777x over previous
import jax
import jax.numpy as jnp
from jax import lax
from jax.experimental import pallas as pl
from jax.experimental.pallas import tpu as pltpu

_ALPHA = 0.2
_BLK_R = 256


def _lrelu(v):
    return jnp.maximum(v, _ALPHA * v)


def _elu(v):
    return jnp.where(v > 0, v, jnp.exp(jnp.minimum(v, 0.0)) - 1.0)


def _stgat_kernel(nhead, nhid, npred,
                  x_ref, adj_ref, wall_ref, a1_ref, a2_ref, a2t_ref,
                  wout_ref, a1ot_ref, a2ot_ref, out_ref,
                  adj_s, va_s, e1_s, e2_s, f1t_s, f2t_s, fb1_s, h_s,
                  vap_s, e1p_s, e2p_s, f1pt_s, f2pt_s, fbp_s):
    ph = pl.program_id(1)
    nb = pl.program_id(2)

    @pl.when(jnp.logical_and(ph == 0, nb == 0))
    def _proj1():
        adj_s[...] = adj_ref[0].astype(jnp.bfloat16)
        wh = jnp.dot(x_ref[0], wall_ref[...],
                     preferred_element_type=jnp.float32)
        wh1 = jnp.dot(wh, a1_ref[...],
                      preferred_element_type=jnp.float32)
        wh2 = jnp.dot(wh, a2_ref[...],
                      preferred_element_type=jnp.float32)
        wh2t = lax.dot_general(
            a2t_ref[...], wh, (((1,), (1,)), ((), ())),
            preferred_element_type=jnp.float32)
        m2 = jnp.max(wh2, axis=0, keepdims=True)
        c = _lrelu(wh1 + m2)
        e1_s[...] = jnp.exp(wh1 - c).astype(jnp.bfloat16)
        e2_s[...] = jnp.exp(_ALPHA * wh1 - c).astype(jnp.bfloat16)
        f1t_s[...] = jnp.exp(wh2t).astype(jnp.bfloat16)
        f2t_s[...] = jnp.exp(_ALPHA * wh2t).astype(jnp.bfloat16)
        ones_col = jnp.ones((wh.shape[0], 1), jnp.bfloat16)
        for h in range(nhead):
            whh = wh[:, h * nhid:(h + 1) * nhid]
            va_s[h, :, :nhid] = whh.astype(jnp.bfloat16)
            va_s[h, :, nhid:nhid + 1] = ones_col
            fb1_s[h:h + 1, :] = jnp.mean(whh, axis=0, keepdims=True)

    @pl.when(jnp.logical_and(ph == 1, nb == 0))
    def _proj2():
        whp = jnp.dot(h_s[...], wout_ref[...],
                      preferred_element_type=jnp.float32)
        wh1p = lax.dot_general(
            whp, a1ot_ref[...], (((1,), (1,)), ((), ())),
            preferred_element_type=jnp.float32)
        wh2pt = lax.dot_general(
            a2ot_ref[...], whp, (((1,), (1,)), ((), ())),
            preferred_element_type=jnp.float32)
        cp = _lrelu(wh1p + jnp.max(wh2pt))
        e1p_s[...] = jnp.exp(wh1p - cp).astype(jnp.bfloat16)
        e2p_s[...] = jnp.exp(_ALPHA * wh1p - cp).astype(jnp.bfloat16)
        f1pt_s[...] = jnp.exp(wh2pt).astype(jnp.bfloat16)
        f2pt_s[...] = jnp.exp(_ALPHA * wh2pt).astype(jnp.bfloat16)
        vap_s[:, :npred] = whp.astype(jnp.bfloat16)
        vap_s[:, npred:npred + 1] = jnp.ones((whp.shape[0], 1), jnp.bfloat16)
        fbp_s[...] = jnp.mean(whp, axis=0, keepdims=True)

    rows = pl.ds(nb * _BLK_R, _BLK_R)
    adjb = adj_s[rows, :]

    @pl.when(ph == 0)
    def _layer1():
        e1b = e1_s[rows, :]
        e2b = e2_s[rows, :]
        outs = []
        for h in range(nhead):
            p = jnp.maximum(e1b[:, h:h + 1] * f1t_s[h:h + 1, :],
                            e2b[:, h:h + 1] * f2t_s[h:h + 1, :])
            w = adjb * p
            na = jnp.dot(w, va_s[h], preferred_element_type=jnp.float32)
            num = na[:, :nhid]
            den = na[:, nhid:nhid + 1]
            hp = jnp.where(den > 0, num / den, fb1_s[h:h + 1, :])
            outs.append(_elu(hp))
        h_s[rows, :] = jnp.concatenate(outs, axis=1)

    @pl.when(ph == 1)
    def _layer2():
        e1b = e1p_s[rows, :]
        e2b = e2p_s[rows, :]
        p = jnp.maximum(e1b * f1pt_s[...], e2b * f2pt_s[...])
        w = adjb * p
        na = jnp.dot(w, vap_s[...], preferred_element_type=jnp.float32)
        num = na[:, :npred]
        den = na[:, npred:npred + 1]
        hp = jnp.where(den > 0, num / den, fbp_s[...])
        out_ref[0] = _elu(hp)


def kernel(x, adj, Ws, a_s, W_out, a_out):
    B, N, F = x.shape
    nhead, _, nhid = Ws.shape
    npred = W_out.shape[1]
    H = nhead * nhid

    wall = jnp.transpose(Ws, (1, 0, 2)).reshape(F, H)
    eye = jnp.eye(nhead, dtype=x.dtype)
    a1 = a_s[:, :nhid, 0]
    a2 = a_s[:, nhid:, 0]
    a1bd = (eye[:, :, None] * a1[None, :, :]).reshape(nhead, H).T
    a2bd = (eye[:, :, None] * a2[None, :, :]).reshape(nhead, H).T
    a2t = a2bd.T
    a1ot = a_out[:npred].T
    a2ot = a_out[npred:].T

    grid = (B, 2, N // _BLK_R)

    def _idx_const(b, ph, nb):
        return (0, 0)

    out = pl.pallas_call(
        lambda *refs: _stgat_kernel(nhead, nhid, npred, *refs),
        grid=grid,
        in_specs=[
            pl.BlockSpec((1, N, F), lambda b, ph, nb: (b, 0, 0)),
            pl.BlockSpec((1, N, N), lambda b, ph, nb: (b, 0, 0)),
            pl.BlockSpec((F, H), _idx_const),
            pl.BlockSpec((H, nhead), _idx_const),
            pl.BlockSpec((H, nhead), _idx_const),
            pl.BlockSpec((nhead, H), _idx_const),
            pl.BlockSpec((H, npred), _idx_const),
            pl.BlockSpec((1, npred), _idx_const),
            pl.BlockSpec((1, npred), _idx_const),
        ],
        out_specs=pl.BlockSpec((1, _BLK_R, npred), lambda b, ph, nb: (b, nb, 0)),
        out_shape=jax.ShapeDtypeStruct((B, N, npred), jnp.float32),
        scratch_shapes=[
            pltpu.VMEM((N, N), jnp.bfloat16),
            pltpu.VMEM((nhead, N, nhid + 1), jnp.bfloat16),
            pltpu.VMEM((N, nhead), jnp.bfloat16),
            pltpu.VMEM((N, nhead), jnp.bfloat16),
            pltpu.VMEM((nhead, N), jnp.bfloat16),
            pltpu.VMEM((nhead, N), jnp.bfloat16),
            pltpu.VMEM((nhead, nhid), jnp.float32),
            pltpu.VMEM((N, H), jnp.float32),
            pltpu.VMEM((N, npred + 1), jnp.bfloat16),
            pltpu.VMEM((N, 1), jnp.bfloat16),
            pltpu.VMEM((N, 1), jnp.bfloat16),
            pltpu.VMEM((1, N), jnp.bfloat16),
            pltpu.VMEM((1, N), jnp.bfloat16),
            pltpu.VMEM((1, npred), jnp.float32),
        ],
    )(x, adj, wall, a1bd, a2bd, a2t, W_out, a1ot, a2ot)
    return out.reshape(B, N * npred)
